# baseline (speedup 1.0000x reference)
import jax
import jax.numpy as jnp
from jax.experimental import pallas as pl
from jax.experimental.pallas import tpu as pltpu

_BLOCK_ROWS = 25000
_WIDE = 128


def _copy_block(in_ref, out_ref):
    out_ref[...] = in_ref[...]


def kernel(inputs, p_emb):
    n, d = p_emb.shape
    wide = p_emb.reshape(n * d // _WIDE, _WIDE)
    rows = wide.shape[0]
    grid = rows // _BLOCK_ROWS
    out = pl.pallas_call(
        _copy_block,
        grid=(grid,),
        in_specs=[pl.BlockSpec((_BLOCK_ROWS, _WIDE), lambda i: (i, 0))],
        out_specs=pl.BlockSpec((_BLOCK_ROWS, _WIDE), lambda i: (i, 0)),
        out_shape=jax.ShapeDtypeStruct(wide.shape, wide.dtype),
    )(wide)
    return out.reshape(n, d)


# speedup vs baseline: 8.6627x; 8.6627x over previous
import jax
import jax.numpy as jnp
from jax.experimental import pallas as pl
from jax.experimental.pallas import tpu as pltpu

_BLOCK_COLS = 32768


def _copy_block(in_ref, out_ref):
    out_ref[...] = in_ref[...]


def kernel(inputs, p_emb):
    n, d = p_emb.shape
    t = p_emb.T
    grid = pl.cdiv(n, _BLOCK_COLS)
    out = pl.pallas_call(
        _copy_block,
        grid=(grid,),
        in_specs=[pl.BlockSpec((d, _BLOCK_COLS), lambda i: (0, i))],
        out_specs=pl.BlockSpec((d, _BLOCK_COLS), lambda i: (0, i)),
        out_shape=jax.ShapeDtypeStruct(t.shape, t.dtype),
    )(t)
    return out.T
